# baseline (speedup 1.0000x reference)
import functools

import jax
import jax.numpy as jnp
from jax import lax
from jax.experimental import pallas as pl
from jax.experimental.pallas import tpu as pltpu
from jax.experimental.pallas import tpu_sc as plsc

AUG_THRESHOLD = 1024
KTC = 11
BIG16 = 32767
NBUF_IN = 4

CPOS = 128
NB = 4
CW = CPOS * 128
NCAND = 13


def _make_tc_outer(R, L, D):
    def outer(lens_ref, ptab_ref, x_hbm, o_hbm):
        def inner(x_ref, o_ref):
            b = pl.program_id(0)
            ln = lens_ref[b]
            lim = jnp.where(ln > AUG_THRESHOLD, ln, 0).astype(jnp.int16)
            o_ref[0] = jnp.where(ptab_ref[0] < lim, 0.0, x_ref[0])

        pipeline = pltpu.emit_pipeline(
            inner,
            grid=(R,),
            in_specs=[
                pl.BlockSpec(
                    (1, L, D),
                    lambda g: (g, 0, 0),
                    pipeline_mode=pl.Buffered(
                        buffer_count=NBUF_IN, use_lookahead=True
                    ),
                )
            ],
            out_specs=[pl.BlockSpec((1, L, D), lambda g: (g, 0, 0))],
        )
        pipeline(x_hbm, o_hbm)

    return outer


def _tc_kernel(x, seq_lens):
    R, L, D = x.shape
    pos = jnp.arange(L, dtype=jnp.int32)
    ptab = jnp.where(pos % 10 == 0, pos, BIG16).astype(jnp.int16)[None, :, None]
    return pl.pallas_call(
        _make_tc_outer(R, L, D),
        grid_spec=pltpu.PrefetchScalarGridSpec(
            num_scalar_prefetch=1,
            grid=(1,),
            in_specs=[
                pl.BlockSpec(memory_space=pltpu.VMEM),
                pl.BlockSpec(memory_space=pltpu.HBM),
            ],
            out_specs=pl.BlockSpec(memory_space=pltpu.HBM),
        ),
        out_shape=jax.ShapeDtypeStruct((R, L, D), x.dtype),
    )(seq_lens, ptab, x)


def _sc_body(row0, nr, L, D, x_hbm, lens_hbm, o_hbm, b0, b1, b2, b3, lens_v,
             sem_in, sem_out):
    bufs = (b0, b1, b2, b3)
    nchunk = nr
    c = lax.axis_index("c")
    s = lax.axis_index("s")
    w = s * 2 + c

    pltpu.sync_copy(lens_hbm, lens_v)

    def chunk_pos(g):
        return (w * nchunk + g) * CPOS

    def in_copy(g, j):
        off = chunk_pos(g) * D
        return pltpu.make_async_copy(
            x_hbm.at[pl.ds(off, CW)], bufs[j], sem_in.at[j]
        )

    def out_copy(g, j):
        off = chunk_pos(g) * D
        return pltpu.make_async_copy(
            bufs[j], o_hbm.at[pl.ds(off, CW)], sem_out.at[j]
        )

    zeros16 = jnp.zeros((16,), jnp.float32)

    for g in range(min(NB, nchunk)):
        in_copy(g, g % NB).start()

    for g in range(nchunk):
        j = g % NB
        if g >= 2 and g + 2 < nchunk:
            out_copy(g - 2, (g - 2) % NB).wait()
            in_copy(g + 2, (g + 2) % NB).start()
        in_copy(g, j).wait()

        gpos = chunk_pos(g)
        row = row0 + gpos // L
        base = gpos - (gpos // L) * L
        row_idx = jnp.full((16,), row, jnp.int32)
        ln_vec = plsc.load_gather(lens_v, [row_idx])
        lim_vec = jnp.where(ln_vec > AUG_THRESHOLD, ln_vec, 0)
        first = lax.rem(10 - lax.rem(base, 10), 10)
        buf = bufs[j]

        def zero_body(i, _):
            offc = first + 10 * i
            p_vec = jnp.full((16,), base + offc, jnp.int32)
            off_vec = jnp.full((16,), offc, jnp.int32)
            cond = (off_vec < CPOS) & (p_vec < lim_vec)
            addr = jnp.minimum(offc, CPOS - 1) * D
            for k in range(8):
                sl = pl.ds(addr + 16 * k, 16)
                buf[sl] = jnp.where(cond, zeros16, buf[sl])
            return 0

        lax.fori_loop(0, NCAND, zero_body, 0)

        out_copy(g, j).start()

    for g in range(max(nchunk - NB, 0), nchunk):
        out_copy(g, g % NB).wait()


def _sc_kernel(x, seq_lens, row0):
    nr, L, D = x.shape
    x1 = x.reshape(-1)
    mesh = plsc.VectorSubcoreMesh(core_axis_name="c", subcore_axis_name="s")
    kern = functools.partial(
        pl.kernel,
        mesh=mesh,
        out_type=jax.ShapeDtypeStruct((nr * L * D,), jnp.float32),
        scratch_types=[
            pltpu.VMEM((CW,), jnp.float32),
            pltpu.VMEM((CW,), jnp.float32),
            pltpu.VMEM((CW,), jnp.float32),
            pltpu.VMEM((CW,), jnp.float32),
            pltpu.VMEM((16,), jnp.int32),
            pltpu.SemaphoreType.DMA((NB,)),
            pltpu.SemaphoreType.DMA((NB,)),
        ],
        compiler_params=pltpu.CompilerParams(needs_layout_passes=False),
    )(functools.partial(_sc_body, row0, nr, L, D))
    return kern(x1, seq_lens).reshape(nr, L, D)


def kernel(sequences, seq_lens):
    B, L, D = sequences.shape
    tc_out = _tc_kernel(sequences[:KTC], seq_lens)
    sc_out = _sc_kernel(sequences[KTC:], seq_lens, KTC)
    return jnp.concatenate([tc_out, sc_out], axis=0), seq_lens


# speedup vs baseline: 3.5192x; 3.5192x over previous
import jax
import jax.numpy as jnp
from jax.experimental import pallas as pl
from jax.experimental.pallas import tpu as pltpu

AUG_THRESHOLD = 1024
BIG16 = 32767
NBUF_IN = 4
BR = 4


def _make_outer(B, L, D):
    def outer(lens_ref, ptab_ref, x_hbm, o_hbm):
        def inner(x_ref, o_ref):
            g = pl.program_id(0)
            ptab = ptab_ref[0]
            for j in range(BR):
                ln = lens_ref[g * BR + j]
                lim = jnp.where(ln > AUG_THRESHOLD, ln, 0).astype(jnp.int16)
                o_ref[j] = jnp.where(ptab < lim, 0.0, x_ref[j])

        pipeline = pltpu.emit_pipeline(
            inner,
            grid=(B // BR,),
            in_specs=[
                pl.BlockSpec(
                    (BR, L, D),
                    lambda g: (g, 0, 0),
                    pipeline_mode=pl.Buffered(
                        buffer_count=NBUF_IN, use_lookahead=True
                    ),
                )
            ],
            out_specs=[pl.BlockSpec((BR, L, D), lambda g: (g, 0, 0))],
        )
        pipeline(x_hbm, o_hbm)

    return outer


def kernel(sequences, seq_lens):
    B, L, D = sequences.shape
    pos = jnp.arange(L, dtype=jnp.int32)
    ptab = jnp.where(pos % 10 == 0, pos, BIG16).astype(jnp.int16)[None, :, None]
    out = pl.pallas_call(
        _make_outer(B, L, D),
        grid_spec=pltpu.PrefetchScalarGridSpec(
            num_scalar_prefetch=1,
            grid=(1,),
            in_specs=[
                pl.BlockSpec(memory_space=pltpu.VMEM),
                pl.BlockSpec(memory_space=pltpu.HBM),
            ],
            out_specs=pl.BlockSpec(memory_space=pltpu.HBM),
        ),
        out_shape=jax.ShapeDtypeStruct((B, L, D), sequences.dtype),
    )(seq_lens, ptab, sequences)
    return out, seq_lens
